# baseline (speedup 1.0000x reference)
import functools

import jax
import jax.numpy as jnp
from jax import lax
from jax.experimental import pallas as pl
from jax.experimental.pallas import tpu as pltpu

M = 32
K = 256
D = 4
F = 128
B = 16384
BB = 256


def _pq_body(x_ref, cb_ref, cbt_ref, xhat_ref, codes_ref):
    x = x_ref[...]
    r = lax.broadcasted_iota(jnp.int32, (F, F), 0) // D
    c = lax.broadcasted_iota(jnp.int32, (F, F), 1) // D
    g = (r == c).astype(jnp.float32)
    ssq = jnp.dot(x * x, g, preferred_element_type=jnp.float32,
                  precision=lax.Precision.HIGHEST)
    inv = lax.rsqrt(jnp.maximum(ssq, 1e-24))
    xn = x * inv
    ips = jnp.dot(xn, cb_ref[...], preferred_element_type=jnp.float32,
                  precision=lax.Precision.HIGHEST)
    e = jnp.exp(ips)
    sr = lax.broadcasted_iota(jnp.int32, (M * K, M), 0) // K
    sc = lax.broadcasted_iota(jnp.int32, (M * K, M), 1)
    seg = (sr == sc).astype(jnp.float32)
    s = jnp.dot(e, seg, preferred_element_type=jnp.float32,
                precision=lax.Precision.HIGHEST)
    sinv = 1.0 / s
    sb = jnp.dot(sinv, seg.T, preferred_element_type=jnp.float32,
                 precision=lax.Precision.HIGHEST)
    codes = e * sb
    codes_ref[...] = codes
    xhat_ref[...] = jnp.dot(codes, cbt_ref[...],
                            preferred_element_type=jnp.float32,
                            precision=lax.Precision.HIGHEST)


def kernel(x, C):
    ct = jnp.transpose(C, (0, 2, 1))
    eye = jnp.eye(M, dtype=jnp.float32)
    cb = (eye[:, None, :, None] * ct[:, :, None, :]).reshape(F, M * K)
    cbt = cb.T
    grid = (B // BB,)
    xhat, codes = pl.pallas_call(
        _pq_body,
        grid=grid,
        in_specs=[
            pl.BlockSpec((BB, F), lambda i: (i, 0)),
            pl.BlockSpec((F, M * K), lambda i: (0, 0)),
            pl.BlockSpec((M * K, F), lambda i: (0, 0)),
        ],
        out_specs=[
            pl.BlockSpec((BB, F), lambda i: (i, 0)),
            pl.BlockSpec((BB, M * K), lambda i: (i, 0)),
        ],
        out_shape=[
            jax.ShapeDtypeStruct((B, F), jnp.float32),
            jax.ShapeDtypeStruct((B, M * K), jnp.float32),
        ],
    )(x, cb, cbt)
    return xhat, codes.reshape(B, M, K)


# speedup vs baseline: 2.9544x; 2.9544x over previous
import functools

import jax
import jax.numpy as jnp
from jax import lax
from jax.experimental import pallas as pl
from jax.experimental.pallas import tpu as pltpu

M = 32
K = 256
D = 4
F = 128
B = 16384
BB = 256


def _pq_body(x_ref, cb_ref, cbt_ref, xhat_ref, codes_ref):
    x = x_ref[...]
    r = lax.broadcasted_iota(jnp.int32, (F, F), 0) // D
    c = lax.broadcasted_iota(jnp.int32, (F, F), 1) // D
    g = (r == c).astype(jnp.float32)
    ssq = jnp.dot(x * x, g, preferred_element_type=jnp.float32,
                  precision=lax.Precision.HIGHEST)
    inv = lax.rsqrt(jnp.maximum(ssq, 1e-24))
    xn = x * inv
    ips = jnp.dot(xn, cb_ref[...], preferred_element_type=jnp.float32,
                  precision=lax.Precision.DEFAULT)
    e = jnp.exp(ips)
    sr = lax.broadcasted_iota(jnp.int32, (M * K, M), 0) // K
    sc = lax.broadcasted_iota(jnp.int32, (M * K, M), 1)
    seg = (sr == sc).astype(jnp.float32)
    s = jnp.dot(e, seg, preferred_element_type=jnp.float32,
                precision=lax.Precision.DEFAULT)
    sinv = 1.0 / s
    sb = jnp.dot(sinv, seg.T, preferred_element_type=jnp.float32,
                 precision=lax.Precision.DEFAULT)
    codes = e * sb
    codes_ref[...] = codes
    xhat_ref[...] = jnp.dot(codes, cbt_ref[...],
                            preferred_element_type=jnp.float32,
                            precision=lax.Precision.DEFAULT)


def kernel(x, C):
    ct = jnp.transpose(C, (0, 2, 1))
    eye = jnp.eye(M, dtype=jnp.float32)
    cb = (eye[:, None, :, None] * ct[:, :, None, :]).reshape(F, M * K)
    cbt = cb.T
    grid = (B // BB,)
    xhat, codes = pl.pallas_call(
        _pq_body,
        grid=grid,
        in_specs=[
            pl.BlockSpec((BB, F), lambda i: (i, 0)),
            pl.BlockSpec((F, M * K), lambda i: (0, 0)),
            pl.BlockSpec((M * K, F), lambda i: (0, 0)),
        ],
        out_specs=[
            pl.BlockSpec((BB, F), lambda i: (i, 0)),
            pl.BlockSpec((BB, M * K), lambda i: (i, 0)),
        ],
        out_shape=[
            jax.ShapeDtypeStruct((B, F), jnp.float32),
            jax.ShapeDtypeStruct((B, M * K), jnp.float32),
        ],
    )(x, cb, cbt)
    return xhat, codes.reshape(B, M, K)


# speedup vs baseline: 2.9885x; 1.0115x over previous
import functools

import jax
import jax.numpy as jnp
from jax import lax
from jax.experimental import pallas as pl
from jax.experimental.pallas import tpu as pltpu

M = 32
K = 256
D = 4
F = 128
B = 16384
BB = 256


def _pq_body(x_ref, cb_ref, cbt_ref, seg_ref, segt_ref, xhat_ref, codes_ref):
    x = x_ref[...]
    r = lax.broadcasted_iota(jnp.int32, (F, F), 0) // D
    c = lax.broadcasted_iota(jnp.int32, (F, F), 1) // D
    g = (r == c).astype(jnp.float32)
    ssq = jnp.dot(x * x, g, preferred_element_type=jnp.float32,
                  precision=lax.Precision.HIGHEST)
    inv = lax.rsqrt(jnp.maximum(ssq, 1e-24))
    xn = (x * inv).astype(jnp.bfloat16)
    ips = jnp.dot(xn, cb_ref[...],
                  preferred_element_type=jnp.float32)
    e = jnp.exp(ips)
    eh = e.astype(jnp.bfloat16)
    s = jnp.dot(eh, seg_ref[...], preferred_element_type=jnp.float32)
    sinv = (1.0 / s).astype(jnp.bfloat16)
    sb = jnp.dot(sinv, segt_ref[...],
                 preferred_element_type=jnp.float32)
    codes = e * sb
    codes_ref[...] = codes
    xhat_ref[...] = jnp.dot(codes.astype(jnp.bfloat16), cbt_ref[...],
                            preferred_element_type=jnp.float32)


def kernel(x, C):
    ct = jnp.transpose(C, (0, 2, 1))
    eye = jnp.eye(M, dtype=jnp.float32)
    cb = (eye[:, None, :, None] * ct[:, :, None, :]).reshape(F, M * K)
    cbt = cb.T.astype(jnp.bfloat16)
    cb = cb.astype(jnp.bfloat16)
    gi = jnp.arange(M * K, dtype=jnp.int32) // K
    seg = (gi[:, None] == jnp.arange(M, dtype=jnp.int32)[None, :])
    seg = seg.astype(jnp.bfloat16)
    segt = seg.T
    grid = (B // BB,)
    xhat, codes = pl.pallas_call(
        _pq_body,
        grid=grid,
        in_specs=[
            pl.BlockSpec((BB, F), lambda i: (i, 0)),
            pl.BlockSpec((F, M * K), lambda i: (0, 0)),
            pl.BlockSpec((M * K, F), lambda i: (0, 0)),
            pl.BlockSpec((M * K, M), lambda i: (0, 0)),
            pl.BlockSpec((M, M * K), lambda i: (0, 0)),
        ],
        out_specs=[
            pl.BlockSpec((BB, F), lambda i: (i, 0)),
            pl.BlockSpec((BB, M * K), lambda i: (i, 0)),
        ],
        out_shape=[
            jax.ShapeDtypeStruct((B, F), jnp.float32),
            jax.ShapeDtypeStruct((B, M * K), jnp.float32),
        ],
    )(x, cb, cbt, seg, segt)
    return xhat, codes.reshape(B, M, K)


# speedup vs baseline: 4.1755x; 1.3972x over previous
import functools

import jax
import jax.numpy as jnp
from jax import lax
from jax.experimental import pallas as pl
from jax.experimental.pallas import tpu as pltpu

M = 32
K = 256
D = 4
F = 128
B = 16384
BB = 256


def _pq_body(x_ref, cb_ref, cbt_ref, seg_ref, segt_ref, xhat_ref, codes_ref):
    x = x_ref[...]
    r = lax.broadcasted_iota(jnp.int32, (F, F), 0) // D
    c = lax.broadcasted_iota(jnp.int32, (F, F), 1) // D
    g = (r == c).astype(jnp.float32)
    ssq = jnp.dot(x * x, g, preferred_element_type=jnp.float32,
                  precision=lax.Precision.HIGHEST)
    inv = lax.rsqrt(jnp.maximum(ssq, 1e-24))
    xn = (x * inv).astype(jnp.bfloat16)
    ips = jnp.dot(xn, cb_ref[...],
                  preferred_element_type=jnp.float32)
    e = jnp.exp(ips)
    eh = e.astype(jnp.bfloat16)
    s = jnp.dot(eh, seg_ref[...], preferred_element_type=jnp.float32)
    sinv = (1.0 / s).astype(jnp.bfloat16)
    sb = jnp.dot(sinv, segt_ref[...],
                 preferred_element_type=jnp.float32)
    codes = e * sb
    for m in range(M):
        codes_ref[:, m, :] = codes[:, K * m:K * (m + 1)]
    xhat_ref[...] = jnp.dot(codes.astype(jnp.bfloat16), cbt_ref[...],
                            preferred_element_type=jnp.float32)


def kernel(x, C):
    ct = jnp.transpose(C, (0, 2, 1))
    eye = jnp.eye(M, dtype=jnp.float32)
    cb = (eye[:, None, :, None] * ct[:, :, None, :]).reshape(F, M * K)
    cbt = cb.T.astype(jnp.bfloat16)
    cb = cb.astype(jnp.bfloat16)
    gi = jnp.arange(M * K, dtype=jnp.int32) // K
    seg = (gi[:, None] == jnp.arange(M, dtype=jnp.int32)[None, :])
    seg = seg.astype(jnp.bfloat16)
    segt = seg.T
    grid = (B // BB,)
    xhat, codes = pl.pallas_call(
        _pq_body,
        grid=grid,
        in_specs=[
            pl.BlockSpec((BB, F), lambda i: (i, 0)),
            pl.BlockSpec((F, M * K), lambda i: (0, 0)),
            pl.BlockSpec((M * K, F), lambda i: (0, 0)),
            pl.BlockSpec((M * K, M), lambda i: (0, 0)),
            pl.BlockSpec((M, M * K), lambda i: (0, 0)),
        ],
        out_specs=[
            pl.BlockSpec((BB, F), lambda i: (i, 0)),
            pl.BlockSpec((BB, M, K), lambda i: (i, 0, 0)),
        ],
        out_shape=[
            jax.ShapeDtypeStruct((B, F), jnp.float32),
            jax.ShapeDtypeStruct((B, M, K), jnp.float32),
        ],
    )(x, cb, cbt, seg, segt)
    return xhat, codes


# speedup vs baseline: 8.3060x; 1.9892x over previous
import functools

import jax
import jax.numpy as jnp
from jax import lax
from jax.experimental import pallas as pl
from jax.experimental.pallas import tpu as pltpu

M = 32
K = 256
D = 4
F = 128
B = 16384
BB = 256
R = BB * M


def _pq_body(x_ref, cf_ref, cft_ref, mask_ref, xhat_ref, codes_ref):
    x = x_ref[...]
    r = lax.broadcasted_iota(jnp.int32, (F, F), 0) // D
    c = lax.broadcasted_iota(jnp.int32, (F, F), 1) // D
    g = (r == c).astype(jnp.float32)
    ssq = jnp.dot(x * x, g, preferred_element_type=jnp.float32,
                  precision=lax.Precision.HIGHEST)
    inv = lax.rsqrt(jnp.maximum(ssq, 1e-24))
    xn = x * inv
    xrep = jnp.broadcast_to(xn[:, None, :], (BB, M, F)).reshape(R, F)
    xm = (xrep * mask_ref[...]).astype(jnp.bfloat16)
    ips = jnp.dot(xm, cf_ref[...], preferred_element_type=jnp.float32)
    e = jnp.exp(ips)
    s = jnp.dot(e.astype(jnp.bfloat16), jnp.ones((K, 1), jnp.bfloat16),
                preferred_element_type=jnp.float32)
    sb = jnp.broadcast_to(1.0 / s, (R, K))
    codes = e * sb
    codes_ref[...] = codes.reshape(BB, M, K)
    ph = jnp.dot(codes.astype(jnp.bfloat16), cft_ref[...],
                 preferred_element_type=jnp.float32)
    phm = (ph * mask_ref[...]).reshape(BB, M, F)
    xhat_ref[...] = jnp.sum(phm, axis=1)


def kernel(x, C):
    cf = jnp.transpose(C, (0, 2, 1)).reshape(F, K).astype(jnp.bfloat16)
    cft2 = jnp.transpose(C, (1, 0, 2)).reshape(K, F).astype(jnp.bfloat16)
    lane = jnp.arange(F, dtype=jnp.int32) // D
    row = jnp.arange(M, dtype=jnp.int32)
    mask = (lane[None, :] == row[:, None]).astype(jnp.float32)
    mask = jnp.tile(mask, (BB, 1))
    grid = (B // BB,)
    xhat, codes = pl.pallas_call(
        _pq_body,
        grid=grid,
        in_specs=[
            pl.BlockSpec((BB, F), lambda i: (i, 0)),
            pl.BlockSpec((F, K), lambda i: (0, 0)),
            pl.BlockSpec((K, F), lambda i: (0, 0)),
            pl.BlockSpec((R, F), lambda i: (0, 0)),
        ],
        out_specs=[
            pl.BlockSpec((BB, F), lambda i: (i, 0)),
            pl.BlockSpec((BB, M, K), lambda i: (i, 0, 0)),
        ],
        out_shape=[
            jax.ShapeDtypeStruct((B, F), jnp.float32),
            jax.ShapeDtypeStruct((B, M, K), jnp.float32),
        ],
    )(x, cf, cft2, mask)
    return xhat, codes
